# baseline (speedup 1.0000x reference)
import functools

import jax
import jax.numpy as jnp
from jax import lax
from jax.experimental import pallas as pl
from jax.experimental.pallas import tpu as pltpu
from jax.experimental.pallas import tpu_sc as plsc

_NC = 2
_NS = 16
_CH = 128
_D = 16
_NPW = 8
_BW = 256

_SC_PARAMS = pltpu.CompilerParams(use_tc_tiling_on_sc=False)


def _sc_aggregate(e_chunks, np_rows, k):
    ew = _CH * k
    e_per_tile = e_chunks * _CH // (_NC * _NS)
    groups = e_per_tile // ew
    half_groups = groups // 2
    rows_pt = np_rows // _NS
    mesh = plsc.VectorSubcoreMesh(core_axis_name="c", subcore_axis_name="s")

    @functools.partial(
        pl.kernel,
        mesh=mesh,
        out_type=jax.ShapeDtypeStruct((_NC, np_rows, _D), jnp.float32),
        compiler_params=_SC_PARAMS,
        scratch_types=[
            pltpu.VMEM((k * _CH,), jnp.int32),
            pltpu.VMEM((k * _CH,), jnp.int32),
            pltpu.VMEM((k * _CH,), jnp.int32),
            pltpu.VMEM((k * _CH,), jnp.int32),
            pltpu.VMEM((k * _CH, _D), jnp.float32),
            pltpu.VMEM((k * _CH, _D), jnp.float32),
            pltpu.VMEM_SHARED((np_rows, _D), jnp.float32),
            pltpu.SemaphoreType.DMA,
            pltpu.SemaphoreType.DMA,
            pltpu.SemaphoreType.DMA,
        ],
    )
    def kern(feat_hbm, srcc_hbm, dstc_hbm, zeros_hbm, out_hbm,
             src_a, src_b, dst_a, dst_b, rows_a, rows_b, acc,
             sem_a, sem_b, sem_s):
        c = lax.axis_index("c")
        s = lax.axis_index("s")
        r0 = s * rows_pt
        pltpu.sync_copy(zeros_hbm.at[pl.ds(r0, rows_pt), :],
                        acc.at[pl.ds(r0, rows_pt), :])
        plsc.subcore_barrier()
        tile_e0 = (c * _NS + s) * e_per_tile

        def load_and_fire(g, src_v, dst_v, rows_v, sem):
            eb = tile_e0 + g * ew
            pltpu.sync_copy(srcc_hbm.at[pl.ds(eb, ew)], src_v)
            pltpu.sync_copy(dstc_hbm.at[pl.ds(eb, ew)], dst_v)
            pltpu.async_copy(feat_hbm.at[src_v], rows_v, sem)

        def drain_gather(src_v, rows_v, sem):
            pltpu.make_async_copy(feat_hbm.at[src_v], rows_v, sem).wait()

        def scatter(dst_v, rows_v):
            pltpu.async_copy(rows_v, acc.at[dst_v], sem_s, add=True).wait()

        load_and_fire(0, src_a, dst_a, rows_a, sem_a)

        @pl.loop(0, half_groups)
        def _(gg):
            g1 = 2 * gg + 1
            g2 = 2 * gg + 2
            load_and_fire(g1, src_b, dst_b, rows_b, sem_b)
            drain_gather(src_a, rows_a, sem_a)
            scatter(dst_a, rows_a)

            @pl.when(g2 < groups)
            def _():
                load_and_fire(g2, src_a, dst_a, rows_a, sem_a)

            drain_gather(src_b, rows_b, sem_b)
            scatter(dst_b, rows_b)

        plsc.subcore_barrier()
        pltpu.sync_copy(acc.at[pl.ds(r0, rows_pt), :],
                        out_hbm.at[c, pl.ds(r0, rows_pt), :])

    return kern


def _sc_degree(e_chunks, np_rows, k):
    ew = _CH * k
    e_per_tile = e_chunks * _CH // (_NC * _NS)
    groups = e_per_tile // ew
    half_groups = groups // 2
    rows_pt = np_rows // _NS
    mesh = plsc.VectorSubcoreMesh(core_axis_name="c", subcore_axis_name="s")

    @functools.partial(
        pl.kernel,
        mesh=mesh,
        out_type=jax.ShapeDtypeStruct((_NC, np_rows, _D), jnp.float32),
        compiler_params=_SC_PARAMS,
        scratch_types=[
            pltpu.VMEM((k * _CH,), jnp.int32),
            pltpu.VMEM((k * _CH,), jnp.int32),
            pltpu.VMEM((k * _CH, _D), jnp.float32),
            pltpu.VMEM_SHARED((np_rows, _D), jnp.float32),
            pltpu.SemaphoreType.DMA,
            pltpu.SemaphoreType.DMA,
        ],
    )
    def kern(ones_hbm, dstc_hbm, zeros_hbm, out_hbm,
             dst_a, dst_b, ones_v, acc, sem_a, sem_b):
        c = lax.axis_index("c")
        s = lax.axis_index("s")
        r0 = s * rows_pt
        pltpu.sync_copy(ones_hbm, ones_v)
        pltpu.sync_copy(zeros_hbm.at[pl.ds(r0, rows_pt), :],
                        acc.at[pl.ds(r0, rows_pt), :])
        plsc.subcore_barrier()
        tile_e0 = (c * _NS + s) * e_per_tile

        pltpu.sync_copy(dstc_hbm.at[pl.ds(tile_e0, ew)], dst_a)

        @pl.loop(0, half_groups)
        def _(gg):
            g1 = 2 * gg + 1
            g2 = 2 * gg + 2
            eb1 = tile_e0 + g1 * ew
            pltpu.sync_copy(dstc_hbm.at[pl.ds(eb1, ew)], dst_b)
            pltpu.async_copy(ones_v, acc.at[dst_a], sem_a, add=True).wait()

            @pl.when(g2 < groups)
            def _():
                eb2 = tile_e0 + g2 * ew
                pltpu.sync_copy(dstc_hbm.at[pl.ds(eb2, ew)], dst_a)

            pltpu.async_copy(ones_v, acc.at[dst_b], sem_b, add=True).wait()

        plsc.subcore_barrier()
        pltpu.sync_copy(acc.at[pl.ds(r0, rows_pt), :],
                        out_hbm.at[c, pl.ds(r0, rows_pt), :])

    return kern


def _scale_kernel(nw_rows):
    def body(dg, xr, o):
        dinv = lax.rsqrt(dg[0] + dg[1] + 1.0)
        o[...] = xr[...] * dinv

    bsw = lambda: pl.BlockSpec((_BW, 128), lambda i: (i, 0))
    return pl.pallas_call(
        body,
        grid=(nw_rows // _BW,),
        in_specs=[pl.BlockSpec((2, _BW, 128), lambda i: (0, i, 0)), bsw()],
        out_specs=bsw(),
        out_shape=jax.ShapeDtypeStruct((nw_rows, 128), jnp.float32),
    )


def _dense_kernel(nw_rows, n_real):
    def body(s1, xsr, dg, w1, b1r, w2, o):
        dinv = lax.rsqrt(dg[0] + dg[1] + 1.0)
        agg = (s1[0] + s1[1] + xsr[...]) * dinv
        h = jnp.dot(agg, w1[...], preferred_element_type=jnp.float32) + b1r[...]
        h = jnp.maximum(h, 0.0)
        z = jnp.dot(h, w2[...], preferred_element_type=jnp.float32) * dinv
        wr = (lax.broadcasted_iota(jnp.int32, (_BW, 128), 0)
              + pl.program_id(0) * _BW)
        lane = lax.broadcasted_iota(jnp.int32, (_BW, 128), 1)
        nid = wr * _NPW + lane // _D
        o[...] = jnp.where(nid < n_real, z, 0.0)

    bsw = lambda: pl.BlockSpec((_BW, 128), lambda i: (i, 0))
    bs2 = lambda: pl.BlockSpec((2, _BW, 128), lambda i: (0, i, 0))
    return pl.pallas_call(
        body,
        grid=(nw_rows // _BW,),
        in_specs=[bs2(), bsw(), bs2(),
                  pl.BlockSpec((128, 256), lambda i: (0, 0)),
                  pl.BlockSpec((1, 256), lambda i: (0, 0)),
                  pl.BlockSpec((256, 128), lambda i: (0, 0))],
        out_specs=bsw(),
        out_shape=jax.ShapeDtypeStruct((nw_rows, 128), jnp.float32),
    )


def _final_kernel(nw_rows):
    def body(s2, zr, dg, b2r, o):
        dinv = lax.rsqrt(dg[0] + dg[1] + 1.0)
        o[...] = (s2[0] + s2[1] + zr[...]) * dinv + b2r[...]

    bsw = lambda: pl.BlockSpec((_BW, 128), lambda i: (i, 0))
    bs2 = lambda: pl.BlockSpec((2, _BW, 128), lambda i: (0, i, 0))
    return pl.pallas_call(
        body,
        grid=(nw_rows // _BW,),
        in_specs=[bs2(), bsw(), bs2(),
                  pl.BlockSpec((1, 128), lambda i: (0, 0))],
        out_specs=bsw(),
        out_shape=jax.ShapeDtypeStruct((nw_rows, 128), jnp.float32),
    )


def kernel(x, edge_index, W1, b1, W2, b2):
    n = x.shape[0]
    e = edge_index.shape[1]
    f_in = x.shape[1]
    f_mid = W1.shape[1]
    f_out = W2.shape[1]

    group = _NC * _NS * _CH * 8 * 2
    e_pad = ((e + group - 1) // group) * group
    e_chunks = e_pad // _CH
    nodes_per_blk = _BW * _NPW
    np_rows = ((n + 1 + nodes_per_blk - 1) // nodes_per_blk) * nodes_per_blk
    nw_rows = np_rows * _D // 128

    ei = edge_index.astype(jnp.int32)
    pad = jnp.full((e_pad - e,), n, dtype=jnp.int32)
    srcc = jnp.concatenate([ei[0], pad])
    dstc = jnp.concatenate([ei[1], pad])

    x_p = jnp.zeros((np_rows, _D), jnp.float32).at[:n, :f_in].set(x)
    xw = x_p.reshape(nw_rows, 128)
    zeros_nd = jnp.zeros((np_rows, _D), jnp.float32)
    ones_ch = jnp.ones((8 * _CH, _D), jnp.float32)

    w2p = jnp.zeros((f_mid, _D), jnp.float32).at[:, :f_out].set(W2)
    eye8 = jnp.eye(_NPW, dtype=jnp.float32)
    w1bd = jnp.kron(eye8, W1)
    w2bd = jnp.kron(eye8, w2p)
    b1bd = jnp.tile(b1, _NPW).reshape(1, _NPW * f_mid)
    b2p = jnp.zeros((_D,), jnp.float32).at[:f_out].set(b2)
    b2bd = jnp.tile(b2p, _NPW).reshape(1, 128)

    deg = _sc_degree(e_chunks, np_rows, 8)(ones_ch, dstc, zeros_nd)
    degw = deg.reshape(_NC, nw_rows, 128)

    xsw = _scale_kernel(nw_rows)(degw, xw)
    s1 = _sc_aggregate(e_chunks, np_rows, 4)(
        xsw.reshape(np_rows, _D), srcc, dstc, zeros_nd)
    s1w = s1.reshape(_NC, nw_rows, 128)
    zw = _dense_kernel(nw_rows, n)(s1w, xsw, degw, w1bd, b1bd, w2bd)
    s2 = _sc_aggregate(e_chunks, np_rows, 4)(
        zw.reshape(np_rows, _D), srcc, dstc, zeros_nd)
    s2w = s2.reshape(_NC, nw_rows, 128)
    outw = _final_kernel(nw_rows)(s2w, zw, degw, b2bd)
    return outw.reshape(np_rows, _D)[:n, :f_out]


# speedup vs baseline: 1.1147x; 1.1147x over previous
import functools

import jax
import jax.numpy as jnp
from jax import lax
from jax.experimental import pallas as pl
from jax.experimental.pallas import tpu as pltpu
from jax.experimental.pallas import tpu_sc as plsc

_NC = 2
_NS = 16
_CH = 128
_D = 16
_NPW = 8
_BW = 256

_SC_PARAMS = pltpu.CompilerParams(use_tc_tiling_on_sc=False)


def _sc_aggregate(e_chunks, table_rows, d, k):
    ew = _CH * k
    e_per_tile = e_chunks * _CH // (_NC * _NS)
    groups = e_per_tile // ew
    half_groups = groups // 2
    rows_pt = table_rows // _NS
    mesh = plsc.VectorSubcoreMesh(core_axis_name="c", subcore_axis_name="s")

    @functools.partial(
        pl.kernel,
        mesh=mesh,
        out_type=jax.ShapeDtypeStruct((_NC, table_rows, d), jnp.float32),
        compiler_params=_SC_PARAMS,
        scratch_types=[
            pltpu.VMEM((k * _CH,), jnp.int32),
            pltpu.VMEM((k * _CH,), jnp.int32),
            pltpu.VMEM((k * _CH,), jnp.int32),
            pltpu.VMEM((k * _CH,), jnp.int32),
            pltpu.VMEM((k * _CH, d), jnp.float32),
            pltpu.VMEM((k * _CH, d), jnp.float32),
            pltpu.VMEM_SHARED((table_rows, d), jnp.float32),
            pltpu.SemaphoreType.DMA,
            pltpu.SemaphoreType.DMA,
            pltpu.SemaphoreType.DMA,
        ],
    )
    def kern(feat_hbm, srcc_hbm, dstc_hbm, zeros_hbm, out_hbm,
             src_a, src_b, dst_a, dst_b, rows_a, rows_b, acc,
             sem_a, sem_b, sem_s):
        c = lax.axis_index("c")
        s = lax.axis_index("s")
        r0 = s * rows_pt
        pltpu.sync_copy(zeros_hbm.at[pl.ds(r0, rows_pt), :],
                        acc.at[pl.ds(r0, rows_pt), :])
        plsc.subcore_barrier()
        tile_e0 = (c * _NS + s) * e_per_tile

        def load_and_fire(g, src_v, dst_v, rows_v, sem):
            eb = tile_e0 + g * ew
            pltpu.sync_copy(srcc_hbm.at[pl.ds(eb, ew)], src_v)
            pltpu.sync_copy(dstc_hbm.at[pl.ds(eb, ew)], dst_v)
            pltpu.async_copy(feat_hbm.at[src_v], rows_v, sem)

        def drain_gather(src_v, rows_v, sem):
            pltpu.make_async_copy(feat_hbm.at[src_v], rows_v, sem).wait()

        def scatter(dst_v, rows_v):
            pltpu.async_copy(rows_v, acc.at[dst_v], sem_s, add=True).wait()

        load_and_fire(0, src_a, dst_a, rows_a, sem_a)

        @pl.loop(0, half_groups)
        def _(gg):
            g1 = 2 * gg + 1
            g2 = 2 * gg + 2
            load_and_fire(g1, src_b, dst_b, rows_b, sem_b)
            drain_gather(src_a, rows_a, sem_a)
            scatter(dst_a, rows_a)

            @pl.when(g2 < groups)
            def _():
                load_and_fire(g2, src_a, dst_a, rows_a, sem_a)

            drain_gather(src_b, rows_b, sem_b)
            scatter(dst_b, rows_b)

        plsc.subcore_barrier()
        pltpu.sync_copy(acc.at[pl.ds(r0, rows_pt), :],
                        out_hbm.at[c, pl.ds(r0, rows_pt), :])

    return kern


def _sc_degree(e_chunks, table_rows, d, k):
    ew = _CH * k
    e_per_tile = e_chunks * _CH // (_NC * _NS)
    groups = e_per_tile // ew
    half_groups = groups // 2
    rows_pt = table_rows // _NS
    mesh = plsc.VectorSubcoreMesh(core_axis_name="c", subcore_axis_name="s")

    @functools.partial(
        pl.kernel,
        mesh=mesh,
        out_type=jax.ShapeDtypeStruct((_NC, table_rows, d), jnp.float32),
        compiler_params=_SC_PARAMS,
        scratch_types=[
            pltpu.VMEM((k * _CH,), jnp.int32),
            pltpu.VMEM((k * _CH,), jnp.int32),
            pltpu.VMEM((k * _CH, d), jnp.float32),
            pltpu.VMEM_SHARED((table_rows, d), jnp.float32),
            pltpu.SemaphoreType.DMA,
            pltpu.SemaphoreType.DMA,
        ],
    )
    def kern(ones_hbm, dstc_hbm, zeros_hbm, out_hbm,
             dst_a, dst_b, ones_v, acc, sem_a, sem_b):
        c = lax.axis_index("c")
        s = lax.axis_index("s")
        r0 = s * rows_pt
        pltpu.sync_copy(ones_hbm, ones_v)
        pltpu.sync_copy(zeros_hbm.at[pl.ds(r0, rows_pt), :],
                        acc.at[pl.ds(r0, rows_pt), :])
        plsc.subcore_barrier()
        tile_e0 = (c * _NS + s) * e_per_tile

        pltpu.sync_copy(dstc_hbm.at[pl.ds(tile_e0, ew)], dst_a)

        @pl.loop(0, half_groups)
        def _(gg):
            g1 = 2 * gg + 1
            g2 = 2 * gg + 2
            eb1 = tile_e0 + g1 * ew
            pltpu.sync_copy(dstc_hbm.at[pl.ds(eb1, ew)], dst_b)
            pltpu.async_copy(ones_v, acc.at[dst_a], sem_a, add=True).wait()

            @pl.when(g2 < groups)
            def _():
                eb2 = tile_e0 + g2 * ew
                pltpu.sync_copy(dstc_hbm.at[pl.ds(eb2, ew)], dst_a)

            pltpu.async_copy(ones_v, acc.at[dst_b], sem_b, add=True).wait()

        plsc.subcore_barrier()
        pltpu.sync_copy(acc.at[pl.ds(r0, rows_pt), :],
                        out_hbm.at[c, pl.ds(r0, rows_pt), :])

    return kern


def _scale_kernel(nw_rows):
    def body(dg, xr, pr, o):
        ds = dg[0] + dg[1]
        dfull = ds + jnp.dot(ds, pr[...], preferred_element_type=jnp.float32)
        dinv = lax.rsqrt(dfull + 1.0)
        o[...] = xr[...] * dinv

    bsw = lambda: pl.BlockSpec((_BW, 128), lambda i: (i, 0))
    return pl.pallas_call(
        body,
        grid=(nw_rows // _BW,),
        in_specs=[pl.BlockSpec((2, _BW, 128), lambda i: (0, i, 0)), bsw(),
                  pl.BlockSpec((128, 128), lambda i: (0, 0))],
        out_specs=bsw(),
        out_shape=jax.ShapeDtypeStruct((nw_rows, 128), jnp.float32),
    )


def _dense_kernel(nw_rows, n_real):
    def body(s1, xsr, dg, pr, w1, b1r, w2, o):
        ds = dg[0] + dg[1]
        dfull = ds + jnp.dot(ds, pr[...], preferred_element_type=jnp.float32)
        dinv = lax.rsqrt(dfull + 1.0)
        agg = (s1[0] + s1[1] + xsr[...]) * dinv
        h = jnp.dot(agg, w1[...], preferred_element_type=jnp.float32) + b1r[...]
        h = jnp.maximum(h, 0.0)
        z = jnp.dot(h, w2[...], preferred_element_type=jnp.float32) * dinv
        wr = (lax.broadcasted_iota(jnp.int32, (_BW, 128), 0)
              + pl.program_id(0) * _BW)
        lane = lax.broadcasted_iota(jnp.int32, (_BW, 128), 1)
        nid = wr * _NPW + lane // _D
        o[...] = jnp.where(nid < n_real, z, 0.0)

    bsw = lambda: pl.BlockSpec((_BW, 128), lambda i: (i, 0))
    bs2 = lambda: pl.BlockSpec((2, _BW, 128), lambda i: (0, i, 0))
    return pl.pallas_call(
        body,
        grid=(nw_rows // _BW,),
        in_specs=[bs2(), bsw(), bs2(),
                  pl.BlockSpec((128, 128), lambda i: (0, 0)),
                  pl.BlockSpec((128, 256), lambda i: (0, 0)),
                  pl.BlockSpec((1, 256), lambda i: (0, 0)),
                  pl.BlockSpec((256, 128), lambda i: (0, 0))],
        out_specs=bsw(),
        out_shape=jax.ShapeDtypeStruct((nw_rows, 128), jnp.float32),
    )


def _final_kernel(nw_rows):
    def body(s2, zr, dg, pr, b2r, o):
        ds = dg[0] + dg[1]
        dfull = ds + jnp.dot(ds, pr[...], preferred_element_type=jnp.float32)
        dinv = lax.rsqrt(dfull + 1.0)
        o[...] = (s2[0] + s2[1] + zr[...]) * dinv + b2r[...]

    bsw = lambda: pl.BlockSpec((_BW, 128), lambda i: (i, 0))
    bs2 = lambda: pl.BlockSpec((2, _BW, 128), lambda i: (0, i, 0))
    return pl.pallas_call(
        body,
        grid=(nw_rows // _BW,),
        in_specs=[bs2(), bsw(), bs2(),
                  pl.BlockSpec((128, 128), lambda i: (0, 0)),
                  pl.BlockSpec((1, 128), lambda i: (0, 0))],
        out_specs=bsw(),
        out_shape=jax.ShapeDtypeStruct((nw_rows, 128), jnp.float32),
    )


def kernel(x, edge_index, W1, b1, W2, b2):
    n = x.shape[0]
    e = edge_index.shape[1]
    f_in = x.shape[1]
    f_mid = W1.shape[1]
    f_out = W2.shape[1]

    group = _NC * _NS * _CH * 8 * 2
    e_pad = ((e + group - 1) // group) * group
    e_chunks = e_pad // _CH
    nodes_per_blk = _BW * _NPW
    np_rows = ((n + 1 + nodes_per_blk - 1) // nodes_per_blk) * nodes_per_blk
    nw_rows = np_rows * _D // 128

    ei = edge_index.astype(jnp.int32)
    pad = jnp.full((e_pad - e,), n, dtype=jnp.int32)
    srcc = jnp.concatenate([ei[0], pad])
    dstc = jnp.concatenate([ei[1], pad])
    src2 = srcc * 2
    dst2 = dstc * 2

    x_p = jnp.zeros((np_rows, _D), jnp.float32).at[:n, :f_in].set(x)
    xw = x_p.reshape(nw_rows, 128)
    zeros_nd = jnp.zeros((np_rows, _D), jnp.float32)
    ones_ch = jnp.ones((8 * _CH, _D // 2), jnp.float32)

    w2p = jnp.zeros((f_mid, _D), jnp.float32).at[:, :f_out].set(W2)
    eye8 = jnp.eye(_NPW, dtype=jnp.float32)
    w1bd = jnp.kron(eye8, W1)
    w2bd = jnp.kron(eye8, w2p)
    b1bd = jnp.tile(b1, _NPW).reshape(1, _NPW * f_mid)
    b2p = jnp.zeros((_D,), jnp.float32).at[:f_out].set(b2)
    b2bd = jnp.tile(b2p, _NPW).reshape(1, 128)
    swap16 = jnp.zeros((_D, _D), jnp.float32)
    swap16 = swap16.at[jnp.arange(_D), (jnp.arange(_D) + 8) % _D].set(1.0)
    pswap = jnp.kron(eye8, swap16)

    half = _D // 2
    zeros_half = zeros_nd.reshape(2 * np_rows, half)
    deg = _sc_degree(e_chunks, 2 * np_rows, half, 8)(ones_ch, dst2, zeros_half)
    degw = deg.reshape(_NC, nw_rows, 128)

    xsw = _scale_kernel(nw_rows)(degw, xw, pswap)
    s1 = _sc_aggregate(e_chunks, np_rows, _D, 4)(
        xsw.reshape(np_rows, _D), srcc, dstc, zeros_nd)
    s1w = s1.reshape(_NC, nw_rows, 128)
    zw = _dense_kernel(nw_rows, n)(s1w, xsw, degw, pswap, w1bd, b1bd, w2bd)
    s2 = _sc_aggregate(e_chunks, 2 * np_rows, half, 8)(
        zw.reshape(2 * np_rows, half), src2, dst2, zeros_half)
    s2w = s2.reshape(_NC, nw_rows, 128)
    outw = _final_kernel(nw_rows)(s2w, zw, degw, pswap, b2bd)
    return outw.reshape(np_rows, _D)[:n, :f_out]


# speedup vs baseline: 1.1285x; 1.0124x over previous
import functools

import jax
import jax.numpy as jnp
from jax import lax
from jax.experimental import pallas as pl
from jax.experimental.pallas import tpu as pltpu
from jax.experimental.pallas import tpu_sc as plsc

_NC = 2
_NS = 16
_CH = 128
_D = 16
_NPW = 8
_BW = 256

_SC_PARAMS = pltpu.CompilerParams(use_tc_tiling_on_sc=False)


def _sc_aggregate(e_chunks, table_rows, d, k):
    ew = _CH * k
    e_per_tile = e_chunks * _CH // (_NC * _NS)
    groups = e_per_tile // ew
    half_groups = groups // 2
    rows_pt = table_rows // _NS
    mesh = plsc.VectorSubcoreMesh(core_axis_name="c", subcore_axis_name="s")

    @functools.partial(
        pl.kernel,
        mesh=mesh,
        out_type=jax.ShapeDtypeStruct((_NC, table_rows, d), jnp.float32),
        compiler_params=_SC_PARAMS,
        scratch_types=[
            pltpu.VMEM((k * _CH,), jnp.int32),
            pltpu.VMEM((k * _CH,), jnp.int32),
            pltpu.VMEM((k * _CH,), jnp.int32),
            pltpu.VMEM((k * _CH,), jnp.int32),
            pltpu.VMEM((k * _CH, d), jnp.float32),
            pltpu.VMEM((k * _CH, d), jnp.float32),
            pltpu.VMEM_SHARED((table_rows, d), jnp.float32),
            pltpu.SemaphoreType.DMA,
            pltpu.SemaphoreType.DMA,
            pltpu.SemaphoreType.DMA,
        ],
    )
    def kern(feat_hbm, srcc_hbm, dstc_hbm, zeros_hbm, out_hbm,
             src_a, src_b, dst_a, dst_b, rows_a, rows_b, acc,
             sem_a, sem_b, sem_s):
        c = lax.axis_index("c")
        s = lax.axis_index("s")
        r0 = s * rows_pt
        pltpu.sync_copy(zeros_hbm.at[pl.ds(r0, rows_pt), :],
                        acc.at[pl.ds(r0, rows_pt), :])
        plsc.subcore_barrier()
        tile_e0 = (c * _NS + s) * e_per_tile

        def load_and_fire(g, src_v, dst_v, rows_v, sem):
            eb = tile_e0 + g * ew
            pltpu.sync_copy(srcc_hbm.at[pl.ds(eb, ew)], src_v)
            pltpu.sync_copy(dstc_hbm.at[pl.ds(eb, ew)], dst_v)
            pltpu.async_copy(feat_hbm.at[src_v], rows_v, sem)

        def drain_gather(src_v, rows_v, sem):
            pltpu.make_async_copy(feat_hbm.at[src_v], rows_v, sem).wait()

        def scatter(dst_v, rows_v):
            pltpu.async_copy(rows_v, acc.at[dst_v], sem_s, add=True).wait()

        load_and_fire(0, src_a, dst_a, rows_a, sem_a)

        @pl.loop(0, half_groups)
        def _(gg):
            g1 = 2 * gg + 1
            g2 = 2 * gg + 2
            load_and_fire(g1, src_b, dst_b, rows_b, sem_b)
            drain_gather(src_a, rows_a, sem_a)
            scatter(dst_a, rows_a)

            @pl.when(g2 < groups)
            def _():
                load_and_fire(g2, src_a, dst_a, rows_a, sem_a)

            drain_gather(src_b, rows_b, sem_b)
            scatter(dst_b, rows_b)

        plsc.subcore_barrier()
        pltpu.sync_copy(acc.at[pl.ds(r0, rows_pt), :],
                        out_hbm.at[c, pl.ds(r0, rows_pt), :])

    return kern


def _sc_degree(e_chunks, table_rows, d, k):
    ew = _CH * k
    e_per_tile = e_chunks * _CH // (_NC * _NS)
    groups = e_per_tile // ew
    half_groups = groups // 2
    rows_pt = table_rows // _NS
    mesh = plsc.VectorSubcoreMesh(core_axis_name="c", subcore_axis_name="s")

    @functools.partial(
        pl.kernel,
        mesh=mesh,
        out_type=jax.ShapeDtypeStruct((_NC, table_rows, d), jnp.float32),
        compiler_params=_SC_PARAMS,
        scratch_types=[
            pltpu.VMEM((k * _CH,), jnp.int32),
            pltpu.VMEM((k * _CH,), jnp.int32),
            pltpu.VMEM((k * _CH, d), jnp.float32),
            pltpu.VMEM_SHARED((table_rows, d), jnp.float32),
            pltpu.SemaphoreType.DMA,
            pltpu.SemaphoreType.DMA,
        ],
    )
    def kern(ones_hbm, dstc_hbm, zeros_hbm, out_hbm,
             dst_a, dst_b, ones_v, acc, sem_a, sem_b):
        c = lax.axis_index("c")
        s = lax.axis_index("s")
        r0 = s * rows_pt
        pltpu.sync_copy(ones_hbm, ones_v)
        pltpu.sync_copy(zeros_hbm.at[pl.ds(r0, rows_pt), :],
                        acc.at[pl.ds(r0, rows_pt), :])
        plsc.subcore_barrier()
        tile_e0 = (c * _NS + s) * e_per_tile

        pltpu.sync_copy(dstc_hbm.at[pl.ds(tile_e0, ew)], dst_a)

        @pl.loop(0, half_groups)
        def _(gg):
            g1 = 2 * gg + 1
            g2 = 2 * gg + 2
            eb1 = tile_e0 + g1 * ew
            pltpu.sync_copy(dstc_hbm.at[pl.ds(eb1, ew)], dst_b)
            pltpu.async_copy(ones_v, acc.at[dst_a], sem_a, add=True).wait()

            @pl.when(g2 < groups)
            def _():
                eb2 = tile_e0 + g2 * ew
                pltpu.sync_copy(dstc_hbm.at[pl.ds(eb2, ew)], dst_a)

            pltpu.async_copy(ones_v, acc.at[dst_b], sem_b, add=True).wait()

        plsc.subcore_barrier()
        pltpu.sync_copy(acc.at[pl.ds(r0, rows_pt), :],
                        out_hbm.at[c, pl.ds(r0, rows_pt), :])

    return kern


def _scale_kernel(nw_rows):
    def body(dg, xr, o):
        ds = dg[0] + dg[1]
        dfull = ds + jnp.roll(ds, 8, axis=1)
        dinv = lax.rsqrt(dfull + 1.0)
        o[...] = xr[...] * dinv

    bsw = lambda: pl.BlockSpec((_BW, 128), lambda i: (i, 0))
    return pl.pallas_call(
        body,
        grid=(nw_rows // _BW,),
        in_specs=[pl.BlockSpec((2, _BW, 128), lambda i: (0, i, 0)), bsw()],
        out_specs=bsw(),
        out_shape=jax.ShapeDtypeStruct((nw_rows, 128), jnp.float32),
    )


def _dense_kernel(nw_rows, n_real):
    def body(s1, xsr, dg, w1, b1r, w2, o):
        ds = dg[0] + dg[1]
        dfull = ds + jnp.roll(ds, 8, axis=1)
        dinv = lax.rsqrt(dfull + 1.0)
        agg = (s1[0] + s1[1] + xsr[...]) * dinv
        h = jnp.dot(agg, w1[...], preferred_element_type=jnp.float32) + b1r[...]
        h = jnp.maximum(h, 0.0)
        z = jnp.dot(h, w2[...], preferred_element_type=jnp.float32) * dinv
        wr = (lax.broadcasted_iota(jnp.int32, (_BW, 128), 0)
              + pl.program_id(0) * _BW)
        lane = lax.broadcasted_iota(jnp.int32, (_BW, 128), 1)
        nid = wr * _NPW + lane // _D
        o[...] = jnp.where(nid < n_real, z, 0.0)

    bsw = lambda: pl.BlockSpec((_BW, 128), lambda i: (i, 0))
    bs2 = lambda: pl.BlockSpec((2, _BW, 128), lambda i: (0, i, 0))
    return pl.pallas_call(
        body,
        grid=(nw_rows // _BW,),
        in_specs=[bs2(), bsw(), bs2(),
                  pl.BlockSpec((128, 256), lambda i: (0, 0)),
                  pl.BlockSpec((1, 256), lambda i: (0, 0)),
                  pl.BlockSpec((256, 128), lambda i: (0, 0))],
        out_specs=bsw(),
        out_shape=jax.ShapeDtypeStruct((nw_rows, 128), jnp.float32),
    )


def _final_kernel(nw_rows):
    def body(s2, zr, dg, b2r, o):
        ds = dg[0] + dg[1]
        dfull = ds + jnp.roll(ds, 8, axis=1)
        dinv = lax.rsqrt(dfull + 1.0)
        o[...] = (s2[0] + s2[1] + zr[...]) * dinv + b2r[...]

    bsw = lambda: pl.BlockSpec((_BW, 128), lambda i: (i, 0))
    bs2 = lambda: pl.BlockSpec((2, _BW, 128), lambda i: (0, i, 0))
    return pl.pallas_call(
        body,
        grid=(nw_rows // _BW,),
        in_specs=[bs2(), bsw(), bs2(),
                  pl.BlockSpec((1, 128), lambda i: (0, 0))],
        out_specs=bsw(),
        out_shape=jax.ShapeDtypeStruct((nw_rows, 128), jnp.float32),
    )


def kernel(x, edge_index, W1, b1, W2, b2):
    n = x.shape[0]
    e = edge_index.shape[1]
    f_in = x.shape[1]
    f_mid = W1.shape[1]
    f_out = W2.shape[1]

    group = _NC * _NS * _CH * 8 * 2
    e_pad = ((e + group - 1) // group) * group
    e_chunks = e_pad // _CH
    nodes_per_blk = _BW * _NPW
    np_rows = ((n + 1 + nodes_per_blk - 1) // nodes_per_blk) * nodes_per_blk
    nw_rows = np_rows * _D // 128

    ei = edge_index.astype(jnp.int32)
    pad = jnp.full((e_pad - e,), n, dtype=jnp.int32)
    srcc = jnp.concatenate([ei[0], pad])
    dstc = jnp.concatenate([ei[1], pad])
    src2 = srcc * 2
    dst2 = dstc * 2

    x_p = jnp.zeros((np_rows, _D), jnp.float32).at[:n, :f_in].set(x)
    xw = x_p.reshape(nw_rows, 128)
    zeros_nd = jnp.zeros((np_rows, _D), jnp.float32)
    ones_ch = jnp.ones((8 * _CH, _D // 2), jnp.float32)

    w2p = jnp.zeros((f_mid, _D), jnp.float32).at[:, :f_out].set(W2)
    eye8 = jnp.eye(_NPW, dtype=jnp.float32)
    w1bd = jnp.kron(eye8, W1)
    w2bd = jnp.kron(eye8, w2p)
    b1bd = jnp.tile(b1, _NPW).reshape(1, _NPW * f_mid)
    b2p = jnp.zeros((_D,), jnp.float32).at[:f_out].set(b2)
    b2bd = jnp.tile(b2p, _NPW).reshape(1, 128)

    half = _D // 2
    zeros_half = zeros_nd.reshape(2 * np_rows, half)
    deg = _sc_degree(e_chunks, 2 * np_rows, half, 8)(ones_ch, dst2, zeros_half)
    degw = deg.reshape(_NC, nw_rows, 128)

    xsw = _scale_kernel(nw_rows)(degw, xw)
    s1 = _sc_aggregate(e_chunks, np_rows, _D, 4)(
        xsw.reshape(np_rows, _D), srcc, dstc, zeros_nd)
    s1w = s1.reshape(_NC, nw_rows, 128)
    zw = _dense_kernel(nw_rows, n)(s1w, xsw, degw, w1bd, b1bd, w2bd)
    s2 = _sc_aggregate(e_chunks, 2 * np_rows, half, 8)(
        zw.reshape(2 * np_rows, half), src2, dst2, zeros_half)
    s2w = s2.reshape(_NC, nw_rows, 128)
    outw = _final_kernel(nw_rows)(s2w, zw, degw, b2bd)
    return outw.reshape(np_rows, _D)[:n, :f_out]


# speedup vs baseline: 1.1699x; 1.0367x over previous
import functools

import jax
import jax.numpy as jnp
from jax import lax
from jax.experimental import pallas as pl
from jax.experimental.pallas import tpu as pltpu
from jax.experimental.pallas import tpu_sc as plsc

_NC = 2
_NS = 16
_CH = 128
_D = 16
_NPW = 8
_BW = 256

_SC_PARAMS = pltpu.CompilerParams(use_tc_tiling_on_sc=False)


def _sc_aggregate(e_chunks, table_rows, d, k):
    ew = _CH * k
    e_per_tile = e_chunks * _CH // (_NC * _NS)
    groups = e_per_tile // ew
    half_groups = groups // 2
    rows_pt = table_rows // _NS
    mesh = plsc.VectorSubcoreMesh(core_axis_name="c", subcore_axis_name="s")

    @functools.partial(
        pl.kernel,
        mesh=mesh,
        out_type=jax.ShapeDtypeStruct((_NC, table_rows, d), jnp.float32),
        compiler_params=_SC_PARAMS,
        scratch_types=[
            pltpu.VMEM((k * _CH,), jnp.int32),
            pltpu.VMEM((k * _CH,), jnp.int32),
            pltpu.VMEM((k * _CH,), jnp.int32),
            pltpu.VMEM((k * _CH,), jnp.int32),
            pltpu.VMEM((k * _CH, d), jnp.float32),
            pltpu.VMEM((k * _CH, d), jnp.float32),
            pltpu.VMEM_SHARED((table_rows, d), jnp.float32),
            pltpu.SemaphoreType.DMA,
            pltpu.SemaphoreType.DMA,
            pltpu.SemaphoreType.DMA,
        ],
    )
    def kern(feat_hbm, srcc_hbm, dstc_hbm, zeros_hbm, out_hbm,
             src_a, src_b, dst_a, dst_b, rows_a, rows_b, acc,
             sem_a, sem_b, sem_s):
        c = lax.axis_index("c")
        s = lax.axis_index("s")
        r0 = s * rows_pt
        pltpu.sync_copy(zeros_hbm.at[pl.ds(r0, rows_pt), :],
                        acc.at[pl.ds(r0, rows_pt), :])
        plsc.subcore_barrier()
        tile_e0 = (c * _NS + s) * e_per_tile

        def load_and_fire(g, src_v, dst_v, rows_v, sem):
            eb = tile_e0 + g * ew
            pltpu.sync_copy(srcc_hbm.at[pl.ds(eb, ew)], src_v)
            pltpu.sync_copy(dstc_hbm.at[pl.ds(eb, ew)], dst_v)
            pltpu.async_copy(feat_hbm.at[src_v], rows_v, sem)

        def drain_gather(src_v, rows_v, sem):
            pltpu.make_async_copy(feat_hbm.at[src_v], rows_v, sem).wait()

        def scatter(dst_v, rows_v):
            pltpu.async_copy(rows_v, acc.at[dst_v], sem_s, add=True).wait()

        load_and_fire(0, src_a, dst_a, rows_a, sem_a)

        @pl.loop(0, half_groups)
        def _(gg):
            g1 = 2 * gg + 1
            g2 = 2 * gg + 2
            load_and_fire(g1, src_b, dst_b, rows_b, sem_b)
            drain_gather(src_a, rows_a, sem_a)
            scatter(dst_a, rows_a)

            @pl.when(g2 < groups)
            def _():
                load_and_fire(g2, src_a, dst_a, rows_a, sem_a)

            drain_gather(src_b, rows_b, sem_b)
            scatter(dst_b, rows_b)

        plsc.subcore_barrier()
        pltpu.sync_copy(acc.at[pl.ds(r0, rows_pt), :],
                        out_hbm.at[c, pl.ds(r0, rows_pt), :])

    return kern


def _sc_degree(e_chunks, table_rows, d, k):
    ew = _CH * k
    e_per_tile = e_chunks * _CH // (_NC * _NS)
    groups = e_per_tile // ew
    half_groups = groups // 2
    rows_pt = table_rows // _NS
    mesh = plsc.VectorSubcoreMesh(core_axis_name="c", subcore_axis_name="s")

    @functools.partial(
        pl.kernel,
        mesh=mesh,
        out_type=jax.ShapeDtypeStruct((_NC, table_rows, d), jnp.float32),
        compiler_params=_SC_PARAMS,
        scratch_types=[
            pltpu.VMEM((k * _CH,), jnp.int32),
            pltpu.VMEM((k * _CH,), jnp.int32),
            pltpu.VMEM((k * _CH, d), jnp.float32),
            pltpu.VMEM_SHARED((table_rows, d), jnp.float32),
            pltpu.SemaphoreType.DMA,
            pltpu.SemaphoreType.DMA,
        ],
    )
    def kern(ones_hbm, dstc_hbm, zeros_hbm, out_hbm,
             dst_a, dst_b, ones_v, acc, sem_a, sem_b):
        c = lax.axis_index("c")
        s = lax.axis_index("s")
        r0 = s * rows_pt
        pltpu.sync_copy(ones_hbm, ones_v)
        pltpu.sync_copy(zeros_hbm.at[pl.ds(r0, rows_pt), :],
                        acc.at[pl.ds(r0, rows_pt), :])
        plsc.subcore_barrier()
        tile_e0 = (c * _NS + s) * e_per_tile

        pltpu.sync_copy(dstc_hbm.at[pl.ds(tile_e0, ew)], dst_a)

        @pl.loop(0, half_groups)
        def _(gg):
            g1 = 2 * gg + 1
            g2 = 2 * gg + 2
            eb1 = tile_e0 + g1 * ew
            pltpu.sync_copy(dstc_hbm.at[pl.ds(eb1, ew)], dst_b)
            pltpu.async_copy(ones_v, acc.at[dst_a], sem_a, add=True).wait()

            @pl.when(g2 < groups)
            def _():
                eb2 = tile_e0 + g2 * ew
                pltpu.sync_copy(dstc_hbm.at[pl.ds(eb2, ew)], dst_a)

            pltpu.async_copy(ones_v, acc.at[dst_b], sem_b, add=True).wait()

        plsc.subcore_barrier()
        pltpu.sync_copy(acc.at[pl.ds(r0, rows_pt), :],
                        out_hbm.at[c, pl.ds(r0, rows_pt), :])

    return kern


def _scale_kernel(nw_rows):
    def body(dg, xr, o):
        ds = dg[0] + dg[1]
        dfull = ds + jnp.roll(ds, 8, axis=1)
        dinv = lax.rsqrt(dfull + 1.0)
        o[...] = xr[...] * dinv

    bsw = lambda: pl.BlockSpec((_BW, 128), lambda i: (i, 0))
    return pl.pallas_call(
        body,
        grid=(nw_rows // _BW,),
        in_specs=[pl.BlockSpec((2, _BW, 128), lambda i: (0, i, 0)), bsw()],
        out_specs=bsw(),
        out_shape=jax.ShapeDtypeStruct((nw_rows, 128), jnp.float32),
    )


def _dense_kernel(nw_rows, n_real):
    def body(s1, xsr, dg, w1, b1r, w2, o):
        ds = dg[0] + dg[1]
        dfull = ds + jnp.roll(ds, 8, axis=1)
        dinv = lax.rsqrt(dfull + 1.0)
        agg = (s1[0] + s1[1] + xsr[...]) * dinv
        h = jnp.dot(agg, w1[...], preferred_element_type=jnp.float32) + b1r[...]
        h = jnp.maximum(h, 0.0)
        z = jnp.dot(h, w2[...], preferred_element_type=jnp.float32) * dinv
        wr = (lax.broadcasted_iota(jnp.int32, (_BW, 128), 0)
              + pl.program_id(0) * _BW)
        lane = lax.broadcasted_iota(jnp.int32, (_BW, 128), 1)
        nid = wr * _NPW + lane // _D
        o[...] = jnp.where(nid < n_real, z, 0.0)

    bsw = lambda: pl.BlockSpec((_BW, 128), lambda i: (i, 0))
    bs2 = lambda: pl.BlockSpec((2, _BW, 128), lambda i: (0, i, 0))
    return pl.pallas_call(
        body,
        grid=(nw_rows // _BW,),
        in_specs=[bs2(), bsw(), bs2(),
                  pl.BlockSpec((128, 256), lambda i: (0, 0)),
                  pl.BlockSpec((1, 256), lambda i: (0, 0)),
                  pl.BlockSpec((256, 128), lambda i: (0, 0))],
        out_specs=bsw(),
        out_shape=jax.ShapeDtypeStruct((nw_rows, 128), jnp.float32),
    )


def _final_kernel(nw_rows, f_out):
    sel = _NPW * f_out

    def body(s2, zr, dg, b2r, g, o):
        ds = dg[0] + dg[1]
        dfull = ds + jnp.roll(ds, 8, axis=1)
        dinv = lax.rsqrt(dfull + 1.0)
        outw = (s2[0] + s2[1] + zr[...]) * dinv + b2r[...]
        o[...] = jnp.dot(outw, g[...], precision=lax.Precision.HIGHEST,
                         preferred_element_type=jnp.float32)

    bsw = lambda: pl.BlockSpec((_BW, 128), lambda i: (i, 0))
    bs2 = lambda: pl.BlockSpec((2, _BW, 128), lambda i: (0, i, 0))
    return pl.pallas_call(
        body,
        grid=(nw_rows // _BW,),
        in_specs=[bs2(), bsw(), bs2(),
                  pl.BlockSpec((1, 128), lambda i: (0, 0)),
                  pl.BlockSpec((128, sel), lambda i: (0, 0))],
        out_specs=pl.BlockSpec((_BW, sel), lambda i: (i, 0)),
        out_shape=jax.ShapeDtypeStruct((nw_rows, sel), jnp.float32),
    )


def kernel(x, edge_index, W1, b1, W2, b2):
    n = x.shape[0]
    e = edge_index.shape[1]
    f_in = x.shape[1]
    f_mid = W1.shape[1]
    f_out = W2.shape[1]

    group = _NC * _NS * _CH * 8 * 2
    e_pad = ((e + group - 1) // group) * group
    e_chunks = e_pad // _CH
    nodes_per_blk = _BW * _NPW
    np_rows = ((n + 1 + nodes_per_blk - 1) // nodes_per_blk) * nodes_per_blk
    nw_rows = np_rows * _D // 128

    ei = edge_index.astype(jnp.int32)
    pad = jnp.full((e_pad - e,), n, dtype=jnp.int32)
    srcc = jnp.concatenate([ei[0], pad])
    dstc = jnp.concatenate([ei[1], pad])
    src2 = srcc * 2
    dst2 = dstc * 2

    x_p = jnp.zeros((np_rows, _D), jnp.float32).at[:n, :f_in].set(x)
    xw = x_p.reshape(nw_rows, 128)
    zeros_nd = jnp.zeros((np_rows, _D), jnp.float32)
    ones_ch = jnp.ones((8 * _CH, _D // 2), jnp.float32)

    w2p = jnp.zeros((f_mid, _D), jnp.float32).at[:, :f_out].set(W2)
    eye8 = jnp.eye(_NPW, dtype=jnp.float32)
    w1bd = jnp.kron(eye8, W1)
    w2bd = jnp.kron(eye8, w2p)
    b1bd = jnp.tile(b1, _NPW).reshape(1, _NPW * f_mid)
    b2p = jnp.zeros((_D,), jnp.float32).at[:f_out].set(b2)
    b2bd = jnp.tile(b2p, _NPW).reshape(1, 128)

    half = _D // 2
    zeros_half = zeros_nd.reshape(2 * np_rows, half)
    deg = _sc_degree(e_chunks, 2 * np_rows, half, 8)(ones_ch, dst2, zeros_half)
    degw = deg.reshape(_NC, nw_rows, 128)

    xsw = _scale_kernel(nw_rows)(degw, xw)
    s1 = _sc_aggregate(e_chunks, np_rows, _D, 4)(
        xsw.reshape(np_rows, _D), srcc, dstc, zeros_nd)
    s1w = s1.reshape(_NC, nw_rows, 128)
    zw = _dense_kernel(nw_rows, n)(s1w, xsw, degw, w1bd, b1bd, w2bd)
    s2 = _sc_aggregate(e_chunks, 2 * np_rows, half, 8)(
        zw.reshape(2 * np_rows, half), src2, dst2, zeros_half)
    s2w = s2.reshape(_NC, nw_rows, 128)
    gsel = jnp.zeros((128, _NPW * f_out), jnp.float32)
    rows_sel = jnp.arange(_NPW * f_out) // f_out * _D + jnp.arange(_NPW * f_out) % f_out
    gsel = gsel.at[rows_sel, jnp.arange(_NPW * f_out)].set(1.0)
    outn = _final_kernel(nw_rows, f_out)(s2w, zw, degw, b2bd, gsel)
    return outn.reshape(np_rows, f_out)[:n]


# speedup vs baseline: 1.1879x; 1.0154x over previous
import functools

import jax
import jax.numpy as jnp
from jax import lax
from jax.experimental import pallas as pl
from jax.experimental.pallas import tpu as pltpu
from jax.experimental.pallas import tpu_sc as plsc

_NC = 2
_NS = 16
_CH = 128
_D = 16
_NPW = 8
_BW = 448

_SC_PARAMS = pltpu.CompilerParams(use_tc_tiling_on_sc=False)


def _sc_aggregate(e_chunks, table_rows, d, k):
    ew = _CH * k
    e_per_tile = e_chunks * _CH // (_NC * _NS)
    groups = e_per_tile // ew
    half_groups = groups // 2
    rows_pt = table_rows // _NS
    mesh = plsc.VectorSubcoreMesh(core_axis_name="c", subcore_axis_name="s")

    @functools.partial(
        pl.kernel,
        mesh=mesh,
        out_type=jax.ShapeDtypeStruct((_NC, table_rows, d), jnp.float32),
        compiler_params=_SC_PARAMS,
        scratch_types=[
            pltpu.VMEM((k * _CH,), jnp.int32),
            pltpu.VMEM((k * _CH,), jnp.int32),
            pltpu.VMEM((k * _CH,), jnp.int32),
            pltpu.VMEM((k * _CH,), jnp.int32),
            pltpu.VMEM((k * _CH, d), jnp.float32),
            pltpu.VMEM((k * _CH, d), jnp.float32),
            pltpu.VMEM_SHARED((table_rows, d), jnp.float32),
            pltpu.SemaphoreType.DMA,
            pltpu.SemaphoreType.DMA,
            pltpu.SemaphoreType.DMA,
        ],
    )
    def kern(feat_hbm, srcc_hbm, dstc_hbm, zeros_hbm, out_hbm,
             src_a, src_b, dst_a, dst_b, rows_a, rows_b, acc,
             sem_a, sem_b, sem_s):
        c = lax.axis_index("c")
        s = lax.axis_index("s")
        r0 = s * rows_pt
        pltpu.sync_copy(zeros_hbm.at[pl.ds(r0, rows_pt), :],
                        acc.at[pl.ds(r0, rows_pt), :])
        plsc.subcore_barrier()
        tile_e0 = (c * _NS + s) * e_per_tile

        def load_and_fire(g, src_v, dst_v, rows_v, sem):
            eb = tile_e0 + g * ew
            pltpu.sync_copy(srcc_hbm.at[pl.ds(eb, ew)], src_v)
            pltpu.sync_copy(dstc_hbm.at[pl.ds(eb, ew)], dst_v)
            pltpu.async_copy(feat_hbm.at[src_v], rows_v, sem)

        def drain_gather(src_v, rows_v, sem):
            pltpu.make_async_copy(feat_hbm.at[src_v], rows_v, sem).wait()

        def scatter(dst_v, rows_v):
            pltpu.async_copy(rows_v, acc.at[dst_v], sem_s, add=True).wait()

        load_and_fire(0, src_a, dst_a, rows_a, sem_a)

        @pl.loop(0, half_groups)
        def _(gg):
            g1 = 2 * gg + 1
            g2 = 2 * gg + 2
            load_and_fire(g1, src_b, dst_b, rows_b, sem_b)
            drain_gather(src_a, rows_a, sem_a)
            scatter(dst_a, rows_a)

            @pl.when(g2 < groups)
            def _():
                load_and_fire(g2, src_a, dst_a, rows_a, sem_a)

            drain_gather(src_b, rows_b, sem_b)
            scatter(dst_b, rows_b)

        plsc.subcore_barrier()
        pltpu.sync_copy(acc.at[pl.ds(r0, rows_pt), :],
                        out_hbm.at[c, pl.ds(r0, rows_pt), :])

    return kern


def _sc_degree(e_chunks, table_rows, d, k):
    ew = _CH * k
    e_per_tile = e_chunks * _CH // (_NC * _NS)
    groups = e_per_tile // ew
    half_groups = groups // 2
    rows_pt = table_rows // _NS
    mesh = plsc.VectorSubcoreMesh(core_axis_name="c", subcore_axis_name="s")

    @functools.partial(
        pl.kernel,
        mesh=mesh,
        out_type=jax.ShapeDtypeStruct((_NC, table_rows, d), jnp.float32),
        compiler_params=_SC_PARAMS,
        scratch_types=[
            pltpu.VMEM((k * _CH,), jnp.int32),
            pltpu.VMEM((k * _CH,), jnp.int32),
            pltpu.VMEM((k * _CH, d), jnp.float32),
            pltpu.VMEM_SHARED((table_rows, d), jnp.float32),
            pltpu.SemaphoreType.DMA,
            pltpu.SemaphoreType.DMA,
        ],
    )
    def kern(ones_hbm, dstc_hbm, zeros_hbm, out_hbm,
             dst_a, dst_b, ones_v, acc, sem_a, sem_b):
        c = lax.axis_index("c")
        s = lax.axis_index("s")
        r0 = s * rows_pt
        pltpu.sync_copy(ones_hbm, ones_v)
        pltpu.sync_copy(zeros_hbm.at[pl.ds(r0, rows_pt), :],
                        acc.at[pl.ds(r0, rows_pt), :])
        plsc.subcore_barrier()
        tile_e0 = (c * _NS + s) * e_per_tile

        pltpu.sync_copy(dstc_hbm.at[pl.ds(tile_e0, ew)], dst_a)

        @pl.loop(0, half_groups)
        def _(gg):
            g1 = 2 * gg + 1
            g2 = 2 * gg + 2
            eb1 = tile_e0 + g1 * ew
            pltpu.sync_copy(dstc_hbm.at[pl.ds(eb1, ew)], dst_b)
            pltpu.async_copy(ones_v, acc.at[dst_a], sem_a, add=True).wait()

            @pl.when(g2 < groups)
            def _():
                eb2 = tile_e0 + g2 * ew
                pltpu.sync_copy(dstc_hbm.at[pl.ds(eb2, ew)], dst_a)

            pltpu.async_copy(ones_v, acc.at[dst_b], sem_b, add=True).wait()

        plsc.subcore_barrier()
        pltpu.sync_copy(acc.at[pl.ds(r0, rows_pt), :],
                        out_hbm.at[c, pl.ds(r0, rows_pt), :])

    return kern


def _scale_kernel(nw_rows):
    def body(dg, xr, o):
        ds = dg[0] + dg[1]
        dfull = ds + jnp.roll(ds, 8, axis=1)
        dinv = lax.rsqrt(dfull + 1.0)
        o[...] = xr[...] * dinv

    bsw = lambda: pl.BlockSpec((_BW, 128), lambda i: (i, 0))
    return pl.pallas_call(
        body,
        grid=(nw_rows // _BW,),
        in_specs=[pl.BlockSpec((2, _BW, 128), lambda i: (0, i, 0)), bsw()],
        out_specs=bsw(),
        out_shape=jax.ShapeDtypeStruct((nw_rows, 128), jnp.float32),
    )


def _dense_kernel(nw_rows, n_real):
    def body(s1, xsr, dg, w1, b1r, w2, o):
        ds = dg[0] + dg[1]
        dfull = ds + jnp.roll(ds, 8, axis=1)
        dinv = lax.rsqrt(dfull + 1.0)
        agg = (s1[0] + s1[1] + xsr[...]) * dinv
        h = jnp.dot(agg, w1[...], preferred_element_type=jnp.float32) + b1r[...]
        h = jnp.maximum(h, 0.0)
        z = jnp.dot(h, w2[...], preferred_element_type=jnp.float32) * dinv
        wr = (lax.broadcasted_iota(jnp.int32, (_BW, 128), 0)
              + pl.program_id(0) * _BW)
        lane = lax.broadcasted_iota(jnp.int32, (_BW, 128), 1)
        nid = wr * _NPW + lane // _D
        o[...] = jnp.where(nid < n_real, z, 0.0)

    bsw = lambda: pl.BlockSpec((_BW, 128), lambda i: (i, 0))
    bs2 = lambda: pl.BlockSpec((2, _BW, 128), lambda i: (0, i, 0))
    return pl.pallas_call(
        body,
        grid=(nw_rows // _BW,),
        in_specs=[bs2(), bsw(), bs2(),
                  pl.BlockSpec((128, 256), lambda i: (0, 0)),
                  pl.BlockSpec((1, 256), lambda i: (0, 0)),
                  pl.BlockSpec((256, 128), lambda i: (0, 0))],
        out_specs=bsw(),
        out_shape=jax.ShapeDtypeStruct((nw_rows, 128), jnp.float32),
    )


def _final_kernel(nw_rows, f_out):
    sel = _NPW * f_out

    def body(s2, zr, dg, b2r, g, o):
        ds = dg[0] + dg[1]
        dfull = ds + jnp.roll(ds, 8, axis=1)
        dinv = lax.rsqrt(dfull + 1.0)
        outw = (s2[0] + s2[1] + zr[...]) * dinv + b2r[...]
        o[...] = jnp.dot(outw, g[...], precision=lax.Precision.HIGHEST,
                         preferred_element_type=jnp.float32)

    bsw = lambda: pl.BlockSpec((_BW, 128), lambda i: (i, 0))
    bs2 = lambda: pl.BlockSpec((2, _BW, 128), lambda i: (0, i, 0))
    return pl.pallas_call(
        body,
        grid=(nw_rows // _BW,),
        in_specs=[bs2(), bsw(), bs2(),
                  pl.BlockSpec((1, 128), lambda i: (0, 0)),
                  pl.BlockSpec((128, sel), lambda i: (0, 0))],
        out_specs=pl.BlockSpec((_BW, sel), lambda i: (i, 0)),
        out_shape=jax.ShapeDtypeStruct((nw_rows, sel), jnp.float32),
    )


def kernel(x, edge_index, W1, b1, W2, b2):
    n = x.shape[0]
    e = edge_index.shape[1]
    f_in = x.shape[1]
    f_mid = W1.shape[1]
    f_out = W2.shape[1]

    group = _NC * _NS * _CH * 8 * 2
    e_pad = ((e + group - 1) // group) * group
    e_chunks = e_pad // _CH
    nodes_per_blk = _BW * _NPW
    np_rows = ((n + 1 + nodes_per_blk - 1) // nodes_per_blk) * nodes_per_blk
    nw_rows = np_rows * _D // 128

    ei = edge_index.astype(jnp.int32)
    pad = jnp.full((e_pad - e,), n, dtype=jnp.int32)
    srcc = jnp.concatenate([ei[0], pad])
    dstc = jnp.concatenate([ei[1], pad])
    src2 = srcc * 2
    dst2 = dstc * 2

    x_p = jnp.zeros((np_rows, _D), jnp.float32).at[:n, :f_in].set(x)
    xw = x_p.reshape(nw_rows, 128)
    zeros_nd = jnp.zeros((np_rows, _D), jnp.float32)
    ones_ch = jnp.ones((8 * _CH, _D // 2), jnp.float32)

    w2p = jnp.zeros((f_mid, _D), jnp.float32).at[:, :f_out].set(W2)
    eye8 = jnp.eye(_NPW, dtype=jnp.float32)
    w1bd = jnp.kron(eye8, W1)
    w2bd = jnp.kron(eye8, w2p)
    b1bd = jnp.tile(b1, _NPW).reshape(1, _NPW * f_mid)
    b2p = jnp.zeros((_D,), jnp.float32).at[:f_out].set(b2)
    b2bd = jnp.tile(b2p, _NPW).reshape(1, 128)

    half = _D // 2
    zeros_half = zeros_nd.reshape(2 * np_rows, half)
    deg = _sc_degree(e_chunks, 2 * np_rows, half, 8)(ones_ch, dst2, zeros_half)
    degw = deg.reshape(_NC, nw_rows, 128)

    xsw = _scale_kernel(nw_rows)(degw, xw)
    s1 = _sc_aggregate(e_chunks, np_rows, _D, 4)(
        xsw.reshape(np_rows, _D), srcc, dstc, zeros_nd)
    s1w = s1.reshape(_NC, nw_rows, 128)
    zw = _dense_kernel(nw_rows, n)(s1w, xsw, degw, w1bd, b1bd, w2bd)
    s2 = _sc_aggregate(e_chunks, 2 * np_rows, half, 8)(
        zw.reshape(2 * np_rows, half), src2, dst2, zeros_half)
    s2w = s2.reshape(_NC, nw_rows, 128)
    gsel = jnp.zeros((128, _NPW * f_out), jnp.float32)
    rows_sel = jnp.arange(_NPW * f_out) // f_out * _D + jnp.arange(_NPW * f_out) % f_out
    gsel = gsel.at[rows_sel, jnp.arange(_NPW * f_out)].set(1.0)
    outn = _final_kernel(nw_rows, f_out)(s2w, zw, degw, b2bd, gsel)
    return outn.reshape(np_rows, f_out)[:n]
